# baseline (speedup 1.0000x reference)
import functools

import jax
import jax.numpy as jnp
from jax import lax
from jax.experimental import pallas as pl
from jax.experimental.pallas import tpu as pltpu
from jax.experimental.pallas import tpu_sc as plsc

_NUM_WORKERS = 32
_LANES = 16
_RING = 4
_UNROLL = 4


def kernel(tokens, token_embedding, position_embedding):
    B, T = tokens.shape
    V, D = token_embedding.shape
    assert B % _NUM_WORKERS == 0 and T % _RING == 0
    n_rows = B // _NUM_WORKERS
    n_seg = n_rows // _LANES
    d_seg = D // _LANES

    tok_slabs = (
        tokens.reshape(_NUM_WORKERS, n_rows, T)
        .transpose(0, 2, 1)
        .reshape(_NUM_WORKERS * T * n_rows)
    )

    mesh = plsc.VectorSubcoreMesh(core_axis_name="c", subcore_axis_name="s")

    scratch = [
        pltpu.VMEM((T * n_rows,), jnp.int32),
        pltpu.VMEM((T, D), jnp.float32),
        pltpu.VMEM((n_rows,), jnp.int32),
        pltpu.SemaphoreType.DMA,
    ]
    for _ in range(_RING):
        scratch.append(pltpu.VMEM((n_rows,), jnp.int32))
        scratch.append(pltpu.VMEM((n_rows, D), jnp.float32))
        scratch.append(pltpu.SemaphoreType.DMA)
        scratch.append(pltpu.SemaphoreType.DMA)

    @functools.partial(
        pl.kernel,
        mesh=mesh,
        out_type=jax.ShapeDtypeStruct((B * T, D), jnp.float32),
        scratch_types=scratch,
    )
    def emb_kernel(tok_hbm, tab_hbm, pos_hbm, out_hbm, slab, pos_all, ob,
                   sem_p, *scr):
        oix_b = [scr[4 * s + 0] for s in range(_RING)]
        row_b = [scr[4 * s + 1] for s in range(_RING)]
        sem_g = [scr[4 * s + 2] for s in range(_RING)]
        sem_o = [scr[4 * s + 3] for s in range(_RING)]

        wid = lax.axis_index("s") * 2 + lax.axis_index("c")
        b0 = wid * n_rows

        pltpu.async_copy(tok_hbm.at[pl.ds(wid * (T * n_rows), T * n_rows)],
                         slab, sem_p)
        pltpu.async_copy(pos_hbm, pos_all, sem_p)

        for sg in range(n_seg):
            sl = pl.ds(sg * _LANES, _LANES)
            lane = lax.iota(jnp.int32, _LANES) + (sg * _LANES)
            ob[sl] = (lane + b0) * T

        pltpu.make_async_copy(
            tok_hbm.at[pl.ds(wid * (T * n_rows), T * n_rows)], slab, sem_p).wait()
        pltpu.make_async_copy(pos_hbm, pos_all, sem_p).wait()

        def fire_gather(t, s):
            pltpu.async_copy(
                tab_hbm.at[slab.at[pl.ds(t * n_rows, n_rows)]], row_b[s],
                sem_g[s])

        def wait_gather(t, s):
            pltpu.make_async_copy(
                tab_hbm.at[slab.at[pl.ds(t * n_rows, n_rows)]], row_b[s],
                sem_g[s]).wait()

        def fire_scatter(s):
            pltpu.async_copy(row_b[s], out_hbm.at[oix_b[s]], sem_o[s])

        def wait_scatter(s):
            pltpu.make_async_copy(row_b[s], out_hbm.at[oix_b[s]], sem_o[s]).wait()

        def compute(t, s):
            prow = [pos_all[t, pl.ds(k * _LANES, _LANES)] for k in range(d_seg)]
            for sg in range(n_seg):
                sl = pl.ds(sg * _LANES, _LANES)
                oix_b[s][sl] = ob[sl] + t

            def add_rows(r4, c):
                for u in range(_UNROLL):
                    r = r4 * _UNROLL + u
                    for k in range(d_seg):
                        sl = pl.ds(k * _LANES, _LANES)
                        row_b[s][r, sl] = row_b[s][r, sl] + prow[k]
                return c

            lax.fori_loop(0, n_rows // _UNROLL, add_rows, 0)

        fire_gather(0, 0)
        fire_gather(1, 1)

        def body(jo, carry):
            for k in range(_RING):
                t = jo * _RING + k
                s2 = (k + 2) % _RING

                @pl.when(jnp.logical_and(t >= 2, t + 2 < T))
                def _():
                    wait_scatter(s2)

                @pl.when(t + 2 < T)
                def _():
                    fire_gather(t + 2, s2)

                wait_gather(t, k)
                compute(t, k)
                fire_scatter(k)
            return carry

        lax.fori_loop(0, T // _RING, body, 0)
        for k in range(_RING):
            wait_scatter(k)

    out = emb_kernel(tok_slabs, token_embedding, position_embedding)
    return out.reshape(B, T, D)


# speedup vs baseline: 1.0183x; 1.0183x over previous
import functools

import jax
import jax.numpy as jnp
from jax import lax
from jax.experimental import pallas as pl
from jax.experimental.pallas import tpu as pltpu
from jax.experimental.pallas import tpu_sc as plsc

_NUM_WORKERS = 32
_LANES = 16
_RING = 5
_UNROLL = 4


def kernel(tokens, token_embedding, position_embedding):
    B, T = tokens.shape
    V, D = token_embedding.shape
    assert B % _NUM_WORKERS == 0 and T % _RING == 0
    n_rows = B // _NUM_WORKERS
    n_seg = n_rows // _LANES
    d_seg = D // _LANES

    tokens_t = tokens.T

    mesh = plsc.VectorSubcoreMesh(core_axis_name="c", subcore_axis_name="s")

    scratch = []
    for _ in range(_RING):
        scratch.append(pltpu.VMEM((n_rows,), jnp.int32))
        scratch.append(pltpu.VMEM((n_rows,), jnp.float32))
        scratch.append(pltpu.VMEM((n_rows,), jnp.int32))
        scratch.append(pltpu.VMEM((n_rows, D), jnp.float32))
        scratch.append(pltpu.SemaphoreType.DMA)
        scratch.append(pltpu.SemaphoreType.DMA)
        scratch.append(pltpu.SemaphoreType.DMA)
    scratch.append(pltpu.VMEM((n_rows,), jnp.int32))

    @functools.partial(
        pl.kernel,
        mesh=mesh,
        out_type=jax.ShapeDtypeStruct((B * T, D), jnp.float32),
        scratch_types=scratch,
    )
    def emb_kernel(tok_hbm, tab_hbm, pos_hbm, out_hbm, *scr):
        idx_b = [scr[7 * s + 0] for s in range(_RING)]
        pos_b = [scr[7 * s + 1] for s in range(_RING)]
        oix_b = [scr[7 * s + 2] for s in range(_RING)]
        row_b = [scr[7 * s + 3] for s in range(_RING)]
        sem_l = [scr[7 * s + 4] for s in range(_RING)]
        sem_g = [scr[7 * s + 5] for s in range(_RING)]
        sem_o = [scr[7 * s + 6] for s in range(_RING)]
        ob = scr[7 * _RING]

        wid = lax.axis_index("s") * 2 + lax.axis_index("c")
        b0 = wid * n_rows

        for sg in range(n_seg):
            lane = lax.iota(jnp.int32, _LANES) + (sg * _LANES)
            ob[pl.ds(sg * _LANES, _LANES)] = (lane + b0) * T

        def fire_loads(t, s):
            pltpu.async_copy(tok_hbm.at[t, pl.ds(b0, n_rows)], idx_b[s], sem_l[s])
            pltpu.async_copy(pos_hbm.at[t], pos_b[s], sem_l[s])

        def wait_loads(t, s):
            pltpu.make_async_copy(
                tok_hbm.at[t, pl.ds(b0, n_rows)], idx_b[s], sem_l[s]).wait()
            pltpu.make_async_copy(pos_hbm.at[t], pos_b[s], sem_l[s]).wait()

        half = n_rows // 2

        def fire_gather(s):
            pltpu.async_copy(tab_hbm.at[idx_b[s].at[pl.ds(0, half)]],
                             row_b[s].at[pl.ds(0, half)], sem_g[s])
            pltpu.async_copy(tab_hbm.at[idx_b[s].at[pl.ds(half, half)]],
                             row_b[s].at[pl.ds(half, half)], sem_g[s])

        def wait_gather(s):
            pltpu.make_async_copy(tab_hbm.at[idx_b[s].at[pl.ds(0, half)]],
                                  row_b[s].at[pl.ds(0, half)], sem_g[s]).wait()
            pltpu.make_async_copy(tab_hbm.at[idx_b[s].at[pl.ds(half, half)]],
                                  row_b[s].at[pl.ds(half, half)], sem_g[s]).wait()

        def fire_scatter(s):
            pltpu.async_copy(row_b[s], out_hbm.at[oix_b[s]], sem_o[s])

        def wait_scatter(s):
            pltpu.make_async_copy(row_b[s], out_hbm.at[oix_b[s]], sem_o[s]).wait()

        def compute(t, s):
            prow = [pos_b[s][pl.ds(k * _LANES, _LANES)] for k in range(d_seg)]
            for sg in range(n_seg):
                sl = pl.ds(sg * _LANES, _LANES)
                oix_b[s][sl] = ob[sl] + t

            def add_rows(r4, c):
                for u in range(_UNROLL):
                    r = r4 * _UNROLL + u
                    for k in range(d_seg):
                        sl = pl.ds(k * _LANES, _LANES)
                        row_b[s][r, sl] = row_b[s][r, sl] + prow[k]
                return c

            lax.fori_loop(0, n_rows // _UNROLL, add_rows, 0)

        fire_loads(0, 0)
        fire_loads(1, 1)
        fire_loads(2, 2)
        wait_loads(0, 0)
        fire_gather(0)
        wait_loads(1, 1)
        fire_gather(1)

        def body(jo, carry):
            for k in range(_RING):
                t = jo * _RING + k
                s3 = (k + 3) % _RING

                @pl.when(jnp.logical_and(t >= 2, t + 3 < T))
                def _():
                    wait_scatter(s3)

                @pl.when(t + 3 < T)
                def _():
                    fire_loads(t + 3, s3)

                s2 = (k + 2) % _RING

                @pl.when(t + 2 < T)
                def _():
                    wait_loads(t + 2, s2)
                    fire_gather(s2)

                wait_gather(k)
                compute(t, k)
                fire_scatter(k)
            return carry

        lax.fori_loop(0, T // _RING, body, 0)
        for k in range(_RING):
            wait_scatter(k)

    out = emb_kernel(tokens_t, token_embedding, position_embedding)
    return out.reshape(B, T, D)
